# baseline (speedup 1.0000x reference)
import jax
import jax.numpy as jnp
from jax.experimental import pallas as pl
from jax.experimental.pallas import tpu as pltpu

T, H, I2, E, K = 32, 768, 1536, 64, 2
I = I2 // 2
S = K * T
NBUF = 4


def _row_of(col, n):
    i = jax.lax.broadcasted_iota(jnp.int32, (n, n), 0)
    j = jax.lax.broadcasted_iota(jnp.int32, (n, n), 1)
    b = jnp.broadcast_to(col, (n, n))
    return jnp.sum(jnp.where(i == j, b, jnp.zeros_like(b)), axis=0,
                   keepdims=True)


def _router_body(logits_ref, uniq_ref, nu_ref, w_ref):
    logits = logits_ref[...].astype(jnp.float32)
    m = jnp.max(logits, axis=1, keepdims=True)
    p = jnp.exp(logits - m)
    probs = p / jnp.sum(p, axis=1, keepdims=True)

    iota_e = jax.lax.broadcasted_iota(jnp.int32, (T, E), 1)
    m1 = jnp.max(probs, axis=1, keepdims=True)
    i1 = jnp.min(jnp.where(probs == m1, iota_e, E), axis=1, keepdims=True)
    masked = jnp.where(iota_e == i1, -1.0, probs)
    m2 = jnp.max(masked, axis=1, keepdims=True)
    i2 = jnp.min(jnp.where(masked == m2, iota_e, E), axis=1, keepdims=True)

    denom = m1 + m2
    w1 = m1 / denom
    w2 = m2 / denom

    i1r = jnp.broadcast_to(_row_of(i1, T), (E, T))
    i2r = jnp.broadcast_to(_row_of(i2, T), (E, T))
    w1r = jnp.broadcast_to(_row_of(w1, T), (E, T))
    w2r = jnp.broadcast_to(_row_of(w2, T), (E, T))
    e_iota = jax.lax.broadcasted_iota(jnp.int32, (E, T), 0)
    zero = jnp.zeros((E, T), jnp.float32)
    w_ref[...] = (jnp.where(e_iota == i1r, w1r, zero)
                  + jnp.where(e_iota == i2r, w2r, zero))

    e_col = jnp.concatenate([i1, i2], axis=0)
    e_row = _row_of(e_col, S)
    s_col = jax.lax.broadcasted_iota(jnp.int32, (S, S), 0)
    s_row = jax.lax.broadcasted_iota(jnp.int32, (S, S), 1)
    e_colb = jnp.broadcast_to(e_col, (S, S))
    e_rowb = jnp.broadcast_to(e_row, (S, S))
    same = e_rowb == e_colb
    dup_cnt = jnp.sum((same & (s_row < s_col)).astype(jnp.int32), axis=1,
                      keepdims=True)
    first = (dup_cnt == 0).astype(jnp.int32)
    firstb = jnp.broadcast_to(_row_of(first, S), (S, S))
    d = jnp.sum(((firstb == 1) & (e_rowb < e_colb)).astype(jnp.int32),
                axis=1, keepdims=True)
    nu_ref[...] = jnp.sum(first, keepdims=True)
    j_row = jax.lax.broadcasted_iota(jnp.int32, (S, S), 1)
    put = (jnp.broadcast_to(d, (S, S)) == j_row) & (
        jnp.broadcast_to(first, (S, S)) == 1)
    uniq_ref[...] = jnp.sum(jnp.where(put, e_colb, jnp.zeros_like(e_colb)),
                            axis=0, keepdims=True)


def _moe_body(uniq_ref, nu_ref, x_ref, w_ref, gup_ref, dnp_ref, out_ref,
              gbuf, dbuf, gsem, dsem):
    nu = nu_ref[0, 0]
    out_ref[...] = jnp.zeros_like(out_ref)

    def start_copy(u, slot):
        e = uniq_ref[0, u]
        pltpu.make_async_copy(gup_ref.at[pl.ds(e, 1)],
                              gbuf.at[pl.ds(slot, 1)], gsem.at[slot]).start()
        pltpu.make_async_copy(dnp_ref.at[pl.ds(e, 1)],
                              dbuf.at[pl.ds(slot, 1)], dsem.at[slot]).start()

    for b in range(NBUF - 1):
        @pl.when(b < nu)
        def _pro():
            start_copy(b, b)

    def body(u, _):
        nxt = u + NBUF - 1

        @pl.when(nxt < nu)
        def _issue():
            start_copy(nxt, jax.lax.rem(nxt, NBUF))

        slot = jax.lax.rem(u, NBUF)
        e = uniq_ref[0, u]
        pltpu.make_async_copy(gup_ref.at[pl.ds(e, 1)],
                              gbuf.at[pl.ds(slot, 1)], gsem.at[slot]).wait()

        g = gbuf[slot]
        gu = jax.lax.dot_general(
            x_ref[...], g, (((1,), (1,)), ((), ())),
            preferred_element_type=jnp.float32)
        gate = gu[:, :I]
        up = gu[:, I:]
        act = gate * jax.nn.sigmoid(gate) * up

        pltpu.make_async_copy(dnp_ref.at[pl.ds(e, 1)],
                              dbuf.at[pl.ds(slot, 1)], dsem.at[slot]).wait()
        dn = dbuf[slot]
        eo = jax.lax.dot_general(
            act, dn, (((1,), (1,)), ((), ())),
            preferred_element_type=jnp.float32)
        wcol = w_ref[e]
        out_ref[...] += wcol * eo
        return 0

    jax.lax.fori_loop(0, nu, body, 0)


@jax.jit
def _fused_moe(x, router_logits, gate_up_proj, down_proj):
    uniq, nu, wmat = pl.pallas_call(
        _router_body,
        out_shape=(
            jax.ShapeDtypeStruct((1, S), jnp.int32),
            jax.ShapeDtypeStruct((1, 1), jnp.int32),
            jax.ShapeDtypeStruct((E, T), jnp.float32),
        ),
    )(router_logits)

    out = pl.pallas_call(
        _moe_body,
        in_specs=[
            pl.BlockSpec(memory_space=pltpu.SMEM),
            pl.BlockSpec(memory_space=pltpu.SMEM),
            pl.BlockSpec(memory_space=pltpu.VMEM),
            pl.BlockSpec(memory_space=pltpu.VMEM),
            pl.BlockSpec(memory_space=pl.ANY),
            pl.BlockSpec(memory_space=pl.ANY),
        ],
        out_specs=pl.BlockSpec(memory_space=pltpu.VMEM),
        out_shape=jax.ShapeDtypeStruct((T, H), jnp.float32),
        scratch_shapes=[
            pltpu.VMEM((NBUF, I2, H), jnp.float32),
            pltpu.VMEM((NBUF, H, I), jnp.float32),
            pltpu.SemaphoreType.DMA((NBUF,)),
            pltpu.SemaphoreType.DMA((NBUF,)),
        ],
        compiler_params=pltpu.CompilerParams(
            vmem_limit_bytes=100 * 1024 * 1024),
    )(uniq, nu, x, wmat.reshape(E, T, 1), gate_up_proj, down_proj)
    return out


def kernel(x, router_logits, gate_up_proj, down_proj, top_k):
    del top_k
    return _fused_moe(x, router_logits, gate_up_proj, down_proj).astype(x.dtype)


# speedup vs baseline: 1.0060x; 1.0060x over previous
import jax
import jax.numpy as jnp
from jax.experimental import pallas as pl
from jax.experimental.pallas import tpu as pltpu

T, H, I2, E, K = 32, 768, 1536, 64, 2
I = I2 // 2
S = K * T
NBUF = 3


def _row_of(col, n):
    i = jax.lax.broadcasted_iota(jnp.int32, (n, n), 0)
    j = jax.lax.broadcasted_iota(jnp.int32, (n, n), 1)
    b = jnp.broadcast_to(col, (n, n))
    return jnp.sum(jnp.where(i == j, b, jnp.zeros_like(b)), axis=0,
                   keepdims=True)


def _router_body(logits_ref, uniq_ref, nu_ref, w_ref):
    logits = logits_ref[...].astype(jnp.float32)
    m = jnp.max(logits, axis=1, keepdims=True)
    p = jnp.exp(logits - m)
    probs = p / jnp.sum(p, axis=1, keepdims=True)

    iota_e = jax.lax.broadcasted_iota(jnp.int32, (T, E), 1)
    m1 = jnp.max(probs, axis=1, keepdims=True)
    i1 = jnp.min(jnp.where(probs == m1, iota_e, E), axis=1, keepdims=True)
    masked = jnp.where(iota_e == i1, -1.0, probs)
    m2 = jnp.max(masked, axis=1, keepdims=True)
    i2 = jnp.min(jnp.where(masked == m2, iota_e, E), axis=1, keepdims=True)

    denom = m1 + m2
    w1 = m1 / denom
    w2 = m2 / denom

    i1r = jnp.broadcast_to(_row_of(i1, T), (E, T))
    i2r = jnp.broadcast_to(_row_of(i2, T), (E, T))
    w1r = jnp.broadcast_to(_row_of(w1, T), (E, T))
    w2r = jnp.broadcast_to(_row_of(w2, T), (E, T))
    e_iota = jax.lax.broadcasted_iota(jnp.int32, (E, T), 0)
    zero = jnp.zeros((E, T), jnp.float32)
    w_ref[...] = (jnp.where(e_iota == i1r, w1r, zero)
                  + jnp.where(e_iota == i2r, w2r, zero))

    e_col = jnp.concatenate([i1, i2], axis=0)
    e_row = _row_of(e_col, S)
    s_col = jax.lax.broadcasted_iota(jnp.int32, (S, S), 0)
    s_row = jax.lax.broadcasted_iota(jnp.int32, (S, S), 1)
    e_colb = jnp.broadcast_to(e_col, (S, S))
    e_rowb = jnp.broadcast_to(e_row, (S, S))
    same = e_rowb == e_colb
    dup_cnt = jnp.sum((same & (s_row < s_col)).astype(jnp.int32), axis=1,
                      keepdims=True)
    first = (dup_cnt == 0).astype(jnp.int32)
    firstb = jnp.broadcast_to(_row_of(first, S), (S, S))
    d = jnp.sum(((firstb == 1) & (e_rowb < e_colb)).astype(jnp.int32),
                axis=1, keepdims=True)
    nu_ref[...] = jnp.sum(first, keepdims=True)
    j_row = jax.lax.broadcasted_iota(jnp.int32, (S, S), 1)
    put = (jnp.broadcast_to(d, (S, S)) == j_row) & (
        jnp.broadcast_to(first, (S, S)) == 1)
    uniq_ref[...] = jnp.sum(jnp.where(put, e_colb, jnp.zeros_like(e_colb)),
                            axis=0, keepdims=True)


def _moe_body(uniq_ref, nu_ref, x_ref, w_ref, gup_ref, dnp_ref, out_ref,
              gbuf, dbuf, gsem, dsem):
    nu = nu_ref[0, 0]
    out_ref[...] = jnp.zeros_like(out_ref)

    def start_copy(u, slot):
        e = uniq_ref[0, u]
        pltpu.make_async_copy(gup_ref.at[pl.ds(e, 1)],
                              gbuf.at[pl.ds(slot, 1)], gsem.at[slot]).start()
        pltpu.make_async_copy(dnp_ref.at[pl.ds(e, 1)],
                              dbuf.at[pl.ds(slot, 1)], dsem.at[slot]).start()

    for b in range(NBUF - 1):
        @pl.when(b < nu)
        def _pro():
            start_copy(b, b)

    def body(u, _):
        nxt = u + NBUF - 1

        @pl.when(nxt < nu)
        def _issue():
            start_copy(nxt, jax.lax.rem(nxt, NBUF))

        slot = jax.lax.rem(u, NBUF)
        e = uniq_ref[0, u]
        pltpu.make_async_copy(gup_ref.at[pl.ds(e, 1)],
                              gbuf.at[pl.ds(slot, 1)], gsem.at[slot]).wait()

        g = gbuf[slot]
        gu = jax.lax.dot_general(
            x_ref[...], g, (((1,), (1,)), ((), ())),
            preferred_element_type=jnp.float32)
        gate = gu[:, :I]
        up = gu[:, I:]
        act = gate * jax.nn.sigmoid(gate) * up

        pltpu.make_async_copy(dnp_ref.at[pl.ds(e, 1)],
                              dbuf.at[pl.ds(slot, 1)], dsem.at[slot]).wait()
        dn = dbuf[slot]
        eo = jax.lax.dot_general(
            act, dn, (((1,), (1,)), ((), ())),
            preferred_element_type=jnp.float32)
        wcol = w_ref[e]
        out_ref[...] += wcol * eo
        return 0

    jax.lax.fori_loop(0, nu, body, 0)


@jax.jit
def _fused_moe(x, router_logits, gate_up_proj, down_proj):
    uniq, nu, wmat = pl.pallas_call(
        _router_body,
        out_shape=(
            jax.ShapeDtypeStruct((1, S), jnp.int32),
            jax.ShapeDtypeStruct((1, 1), jnp.int32),
            jax.ShapeDtypeStruct((E, T), jnp.float32),
        ),
    )(router_logits)

    out = pl.pallas_call(
        _moe_body,
        in_specs=[
            pl.BlockSpec(memory_space=pltpu.SMEM),
            pl.BlockSpec(memory_space=pltpu.SMEM),
            pl.BlockSpec(memory_space=pltpu.VMEM),
            pl.BlockSpec(memory_space=pltpu.VMEM),
            pl.BlockSpec(memory_space=pl.ANY),
            pl.BlockSpec(memory_space=pl.ANY),
        ],
        out_specs=pl.BlockSpec(memory_space=pltpu.VMEM),
        out_shape=jax.ShapeDtypeStruct((T, H), jnp.float32),
        scratch_shapes=[
            pltpu.VMEM((NBUF, I2, H), jnp.float32),
            pltpu.VMEM((NBUF, H, I), jnp.float32),
            pltpu.SemaphoreType.DMA((NBUF,)),
            pltpu.SemaphoreType.DMA((NBUF,)),
        ],
        compiler_params=pltpu.CompilerParams(
            vmem_limit_bytes=100 * 1024 * 1024),
    )(uniq, nu, x, wmat.reshape(E, T, 1), gate_up_proj, down_proj)
    return out


def kernel(x, router_logits, gate_up_proj, down_proj, top_k):
    del top_k
    return _fused_moe(x, router_logits, gate_up_proj, down_proj).astype(x.dtype)


# speedup vs baseline: 1.0132x; 1.0072x over previous
import jax
import jax.numpy as jnp
from jax.experimental import pallas as pl
from jax.experimental.pallas import tpu as pltpu

T, H, I2, E, K = 32, 768, 1536, 64, 2
I = I2 // 2
S = K * T
NBUF = 3


def _row_of(col, n):
    i = jax.lax.broadcasted_iota(jnp.int32, (n, n), 0)
    j = jax.lax.broadcasted_iota(jnp.int32, (n, n), 1)
    b = jnp.broadcast_to(col, (n, n))
    return jnp.sum(jnp.where(i == j, b, jnp.zeros_like(b)), axis=0,
                   keepdims=True)


def _router_body(logits_ref, uniq_ref, nu_ref, w_ref):
    logits = logits_ref[...].astype(jnp.float32)
    m = jnp.max(logits, axis=1, keepdims=True)
    p = jnp.exp(logits - m)
    probs = p / jnp.sum(p, axis=1, keepdims=True)

    iota_e = jax.lax.broadcasted_iota(jnp.int32, (T, E), 1)
    m1 = jnp.max(probs, axis=1, keepdims=True)
    i1 = jnp.min(jnp.where(probs == m1, iota_e, E), axis=1, keepdims=True)
    masked = jnp.where(iota_e == i1, -1.0, probs)
    m2 = jnp.max(masked, axis=1, keepdims=True)
    i2 = jnp.min(jnp.where(masked == m2, iota_e, E), axis=1, keepdims=True)

    denom = m1 + m2
    w1 = m1 / denom
    w2 = m2 / denom

    i1r = jnp.broadcast_to(_row_of(i1, T), (E, T))
    i2r = jnp.broadcast_to(_row_of(i2, T), (E, T))
    w1r = jnp.broadcast_to(_row_of(w1, T), (E, T))
    w2r = jnp.broadcast_to(_row_of(w2, T), (E, T))
    e_iota = jax.lax.broadcasted_iota(jnp.int32, (E, T), 0)
    zero = jnp.zeros((E, T), jnp.float32)
    w_ref[...] = (jnp.where(e_iota == i1r, w1r, zero)
                  + jnp.where(e_iota == i2r, w2r, zero))

    e_col = jnp.concatenate([i1, i2], axis=0)
    e_row = _row_of(e_col, S)
    s_col = jax.lax.broadcasted_iota(jnp.int32, (S, S), 0)
    s_row = jax.lax.broadcasted_iota(jnp.int32, (S, S), 1)
    e_colb = jnp.broadcast_to(e_col, (S, S))
    e_rowb = jnp.broadcast_to(e_row, (S, S))
    same = e_rowb == e_colb
    dup_cnt = jnp.sum((same & (s_row < s_col)).astype(jnp.int32), axis=1,
                      keepdims=True)
    first = (dup_cnt == 0).astype(jnp.int32)
    firstb = jnp.broadcast_to(_row_of(first, S), (S, S))
    d = jnp.sum(((firstb == 1) & (e_rowb < e_colb)).astype(jnp.int32),
                axis=1, keepdims=True)
    nu_ref[...] = jnp.sum(first, keepdims=True)
    j_row = jax.lax.broadcasted_iota(jnp.int32, (S, S), 1)
    put = (jnp.broadcast_to(d, (S, S)) == j_row) & (
        jnp.broadcast_to(first, (S, S)) == 1)
    uniq_ref[...] = jnp.sum(jnp.where(put, e_colb, jnp.zeros_like(e_colb)),
                            axis=0, keepdims=True)


def _moe_body(uniq_ref, nu_ref, x_ref, w_ref, gup_ref, dnp_ref, out_ref,
              gbuf, dbuf, gsem, dsem):
    nu = nu_ref[0, 0]
    out_ref[...] = jnp.zeros_like(out_ref)

    def start_copy(u, slot):
        e = uniq_ref[0, u]
        pltpu.make_async_copy(gup_ref.at[pl.ds(e, 1)],
                              gbuf.at[pl.ds(slot, 1)], gsem.at[slot]).start()
        pltpu.make_async_copy(dnp_ref.at[pl.ds(e, 1)],
                              dbuf.at[pl.ds(slot, 1)], dsem.at[slot]).start()

    for b in range(NBUF - 1):
        @pl.when(b < nu)
        def _pro():
            start_copy(b, b)

    def body(u, _):
        nxt = u + NBUF - 1

        @pl.when(nxt < nu)
        def _issue():
            start_copy(nxt, jax.lax.rem(nxt, NBUF))

        slot = jax.lax.rem(u, NBUF)
        e = uniq_ref[0, u]
        pltpu.make_async_copy(gup_ref.at[pl.ds(e, 1)],
                              gbuf.at[pl.ds(slot, 1)], gsem.at[slot]).wait()
        pltpu.make_async_copy(dnp_ref.at[pl.ds(e, 1)],
                              dbuf.at[pl.ds(slot, 1)], dsem.at[slot]).wait()

        g = gbuf[slot]
        gu = jax.lax.dot_general(
            x_ref[...], g, (((1,), (1,)), ((), ())),
            preferred_element_type=jnp.float32)
        gate = gu[:, :I]
        up = gu[:, I:]
        act = gate * jax.nn.sigmoid(gate) * up
        dn = dbuf[slot]
        eo = jax.lax.dot_general(
            act, dn, (((1,), (1,)), ((), ())),
            preferred_element_type=jnp.float32)
        wcol = w_ref[e]
        out_ref[...] += wcol * eo
        return 0

    jax.lax.fori_loop(0, nu, body, 0)


@jax.jit
def _fused_moe(x, router_logits, gate_up_proj, down_proj):
    uniq, nu, wmat = pl.pallas_call(
        _router_body,
        out_shape=(
            jax.ShapeDtypeStruct((1, S), jnp.int32),
            jax.ShapeDtypeStruct((1, 1), jnp.int32),
            jax.ShapeDtypeStruct((E, T), jnp.float32),
        ),
    )(router_logits)

    out = pl.pallas_call(
        _moe_body,
        in_specs=[
            pl.BlockSpec(memory_space=pltpu.SMEM),
            pl.BlockSpec(memory_space=pltpu.SMEM),
            pl.BlockSpec(memory_space=pltpu.VMEM),
            pl.BlockSpec(memory_space=pltpu.VMEM),
            pl.BlockSpec(memory_space=pl.ANY),
            pl.BlockSpec(memory_space=pl.ANY),
        ],
        out_specs=pl.BlockSpec(memory_space=pltpu.VMEM),
        out_shape=jax.ShapeDtypeStruct((T, H), jnp.float32),
        scratch_shapes=[
            pltpu.VMEM((NBUF, I2, H), jnp.float32),
            pltpu.VMEM((NBUF, H, I), jnp.float32),
            pltpu.SemaphoreType.DMA((NBUF,)),
            pltpu.SemaphoreType.DMA((NBUF,)),
        ],
        compiler_params=pltpu.CompilerParams(
            vmem_limit_bytes=100 * 1024 * 1024),
    )(uniq, nu, x, wmat.reshape(E, T, 1), gate_up_proj, down_proj)
    return out


def kernel(x, router_logits, gate_up_proj, down_proj, top_k):
    del top_k
    return _fused_moe(x, router_logits, gate_up_proj, down_proj).astype(x.dtype)


# speedup vs baseline: 1.0489x; 1.0352x over previous
import jax
import jax.numpy as jnp
from jax.experimental import pallas as pl
from jax.experimental.pallas import tpu as pltpu

T, H, I2, E, K = 32, 768, 1536, 64, 2
I = I2 // 2
S = K * T
NBUF = 3


def _row_of(col, n):
    i = jax.lax.broadcasted_iota(jnp.int32, (n, n), 0)
    j = jax.lax.broadcasted_iota(jnp.int32, (n, n), 1)
    b = jnp.broadcast_to(col, (n, n))
    return jnp.sum(jnp.where(i == j, b, jnp.zeros_like(b)), axis=0,
                   keepdims=True)


def _col_of(row, n):
    i = jax.lax.broadcasted_iota(jnp.int32, (n, n), 0)
    j = jax.lax.broadcasted_iota(jnp.int32, (n, n), 1)
    b = jnp.broadcast_to(row, (n, n))
    return jnp.sum(jnp.where(i == j, b, jnp.zeros_like(b)), axis=1,
                   keepdims=True)


def _moe_body(logits_ref, x_ref, gup_ref, dnp_ref, out_ref,
              wv, uv, nv, us, ns, gbuf, dbuf, ssem, gsem, dsem):
    logits = logits_ref[...].astype(jnp.float32)
    m = jnp.max(logits, axis=1, keepdims=True)
    p = jnp.exp(logits - m)
    probs = p / jnp.sum(p, axis=1, keepdims=True)

    iota_e = jax.lax.broadcasted_iota(jnp.int32, (T, E), 1)
    m1 = jnp.max(probs, axis=1, keepdims=True)
    i1 = jnp.min(jnp.where(probs == m1, iota_e, E), axis=1, keepdims=True)
    masked = jnp.where(iota_e == i1, -1.0, probs)
    m2 = jnp.max(masked, axis=1, keepdims=True)
    i2 = jnp.min(jnp.where(masked == m2, iota_e, E), axis=1, keepdims=True)

    denom = m1 + m2
    w1 = m1 / denom
    w2 = m2 / denom

    i1r = jnp.broadcast_to(_row_of(i1, T), (E, T))
    i2r = jnp.broadcast_to(_row_of(i2, T), (E, T))
    w1r = jnp.broadcast_to(_row_of(w1, T), (E, T))
    w2r = jnp.broadcast_to(_row_of(w2, T), (E, T))
    e_iota = jax.lax.broadcasted_iota(jnp.int32, (E, T), 0)
    zero = jnp.zeros((E, T), jnp.float32)
    wv[...] = (jnp.where(e_iota == i1r, w1r, zero)
               + jnp.where(e_iota == i2r, w2r, zero))

    e_col = jnp.concatenate([i1, i2], axis=0)
    e_row = _row_of(e_col, S)
    s_col = jax.lax.broadcasted_iota(jnp.int32, (S, S), 0)
    s_row = jax.lax.broadcasted_iota(jnp.int32, (S, S), 1)
    e_colb = jnp.broadcast_to(e_col, (S, S))
    e_rowb = jnp.broadcast_to(e_row, (S, S))
    same = e_rowb == e_colb
    dup_cnt = jnp.sum((same & (s_row < s_col)).astype(jnp.int32), axis=1,
                      keepdims=True)
    first = (dup_cnt == 0).astype(jnp.int32)
    firstb = jnp.broadcast_to(_row_of(first, S), (S, S))
    d = jnp.sum(((firstb == 1) & (e_rowb < e_colb)).astype(jnp.int32),
                axis=1, keepdims=True)
    j_row = jax.lax.broadcasted_iota(jnp.int32, (S, S), 1)
    put = (jnp.broadcast_to(d, (S, S)) == j_row) & (
        jnp.broadcast_to(first, (S, S)) == 1)
    uv[...] = jnp.sum(jnp.where(put, e_colb, jnp.zeros_like(e_colb)),
                      axis=0, keepdims=True)
    nv[...] = jnp.sum(first, keepdims=True)

    pltpu.make_async_copy(uv, us, ssem).start()
    pltpu.make_async_copy(uv, us, ssem).wait()
    pltpu.make_async_copy(nv, ns, ssem).start()
    pltpu.make_async_copy(nv, ns, ssem).wait()

    nu = ns[0, 0]
    out_ref[...] = jnp.zeros_like(out_ref)

    def start_copy(u, slot):
        e = us[0, u]
        pltpu.make_async_copy(gup_ref.at[pl.ds(e, 1)],
                              gbuf.at[pl.ds(slot, 1)], gsem.at[slot]).start()
        pltpu.make_async_copy(dnp_ref.at[pl.ds(e, 1)],
                              dbuf.at[pl.ds(slot, 1)], dsem.at[slot]).start()

    for b in range(NBUF - 1):
        @pl.when(b < nu)
        def _pro():
            start_copy(b, b)

    def body(u, _):
        nxt = u + NBUF - 1

        @pl.when(nxt < nu)
        def _issue():
            start_copy(nxt, jax.lax.rem(nxt, NBUF))

        slot = jax.lax.rem(u, NBUF)
        e = us[0, u]
        pltpu.make_async_copy(gup_ref.at[pl.ds(e, 1)],
                              gbuf.at[pl.ds(slot, 1)], gsem.at[slot]).wait()
        pltpu.make_async_copy(dnp_ref.at[pl.ds(e, 1)],
                              dbuf.at[pl.ds(slot, 1)], dsem.at[slot]).wait()

        g = gbuf[slot]
        gu = jax.lax.dot_general(
            x_ref[...], g, (((1,), (1,)), ((), ())),
            preferred_element_type=jnp.float32)
        gate = gu[:, :I]
        up = gu[:, I:]
        act = gate * jax.nn.sigmoid(gate) * up
        dn = dbuf[slot]
        eo = jax.lax.dot_general(
            act, dn, (((1,), (1,)), ((), ())),
            preferred_element_type=jnp.float32)
        wrow = wv[pl.ds(e, 1), :]
        wcol = _col_of(wrow, T)
        out_ref[...] += wcol * eo
        return 0

    jax.lax.fori_loop(0, nu, body, 0)


@jax.jit
def _fused_moe(x, router_logits, gate_up_proj, down_proj):
    return pl.pallas_call(
        _moe_body,
        in_specs=[
            pl.BlockSpec(memory_space=pltpu.VMEM),
            pl.BlockSpec(memory_space=pltpu.VMEM),
            pl.BlockSpec(memory_space=pl.ANY),
            pl.BlockSpec(memory_space=pl.ANY),
        ],
        out_specs=pl.BlockSpec(memory_space=pltpu.VMEM),
        out_shape=jax.ShapeDtypeStruct((T, H), jnp.float32),
        scratch_shapes=[
            pltpu.VMEM((E, T), jnp.float32),
            pltpu.VMEM((1, S), jnp.int32),
            pltpu.VMEM((1, 1), jnp.int32),
            pltpu.SMEM((1, S), jnp.int32),
            pltpu.SMEM((1, 1), jnp.int32),
            pltpu.VMEM((NBUF, I2, H), jnp.float32),
            pltpu.VMEM((NBUF, H, I), jnp.float32),
            pltpu.SemaphoreType.DMA,
            pltpu.SemaphoreType.DMA((NBUF,)),
            pltpu.SemaphoreType.DMA((NBUF,)),
        ],
        compiler_params=pltpu.CompilerParams(
            vmem_limit_bytes=100 * 1024 * 1024),
    )(router_logits, x, gate_up_proj, down_proj)


def kernel(x, router_logits, gate_up_proj, down_proj, top_k):
    del top_k
    return _fused_moe(x, router_logits, gate_up_proj, down_proj).astype(x.dtype)


# speedup vs baseline: 1.0527x; 1.0037x over previous
import jax
import jax.numpy as jnp
from jax.experimental import pallas as pl
from jax.experimental.pallas import tpu as pltpu

T, H, I2, E, K = 32, 768, 1536, 64, 2
I = I2 // 2
S = K * T
NBUF = 3


def _row_of(col, n):
    i = jax.lax.broadcasted_iota(jnp.int32, (n, n), 0)
    j = jax.lax.broadcasted_iota(jnp.int32, (n, n), 1)
    b = jnp.broadcast_to(col, (n, n))
    return jnp.sum(jnp.where(i == j, b, jnp.zeros_like(b)), axis=0,
                   keepdims=True)


def _col_of(row, n):
    i = jax.lax.broadcasted_iota(jnp.int32, (n, n), 0)
    j = jax.lax.broadcasted_iota(jnp.int32, (n, n), 1)
    b = jnp.broadcast_to(row, (n, n))
    return jnp.sum(jnp.where(i == j, b, jnp.zeros_like(b)), axis=1,
                   keepdims=True)


def _moe_body(logits_ref, x_ref, gup_ref, dnp_ref, out_ref,
              wv, uv, nv, us, ns, gbuf, dbuf, ssem, nsem, gsem, dsem):
    logits = logits_ref[...].astype(jnp.float32)
    m = jnp.max(logits, axis=1, keepdims=True)
    p = jnp.exp(logits - m)
    probs = p / jnp.sum(p, axis=1, keepdims=True)

    iota_e = jax.lax.broadcasted_iota(jnp.int32, (T, E), 1)
    m1 = jnp.max(probs, axis=1, keepdims=True)
    i1 = jnp.min(jnp.where(probs == m1, iota_e, E), axis=1, keepdims=True)
    masked = jnp.where(iota_e == i1, -1.0, probs)
    m2 = jnp.max(masked, axis=1, keepdims=True)
    i2 = jnp.min(jnp.where(masked == m2, iota_e, E), axis=1, keepdims=True)

    denom = m1 + m2
    w1 = m1 / denom
    w2 = m2 / denom

    e_col = jnp.concatenate([i1, i2], axis=0)
    e_row = _row_of(e_col, S)
    s_col = jax.lax.broadcasted_iota(jnp.int32, (S, S), 0)
    s_row = jax.lax.broadcasted_iota(jnp.int32, (S, S), 1)
    e_colb = jnp.broadcast_to(e_col, (S, S))
    e_rowb = jnp.broadcast_to(e_row, (S, S))
    same = e_rowb == e_colb
    dup_cnt = jnp.sum((same & (s_row < s_col)).astype(jnp.int32), axis=1,
                      keepdims=True)
    first = (dup_cnt == 0).astype(jnp.int32)
    firstb = jnp.broadcast_to(_row_of(first, S), (S, S))
    d = jnp.sum(((firstb == 1) & (e_rowb < e_colb)).astype(jnp.int32),
                axis=1, keepdims=True)
    j_row = jax.lax.broadcasted_iota(jnp.int32, (S, S), 1)
    put = (jnp.broadcast_to(d, (S, S)) == j_row) & (
        jnp.broadcast_to(first, (S, S)) == 1)
    uv[...] = jnp.sum(jnp.where(put, e_colb, jnp.zeros_like(e_colb)),
                      axis=0, keepdims=True)
    nv[...] = jnp.sum(first, keepdims=True)

    pltpu.make_async_copy(uv, us, ssem).start()
    pltpu.make_async_copy(nv, ns, nsem).start()
    pltpu.make_async_copy(uv, us, ssem).wait()
    pltpu.make_async_copy(nv, ns, nsem).wait()

    nu = ns[0, 0]

    def start_copy(u, slot):
        e = us[0, u]
        pltpu.make_async_copy(gup_ref.at[pl.ds(e, 1)],
                              gbuf.at[pl.ds(slot, 1)], gsem.at[slot]).start()
        pltpu.make_async_copy(dnp_ref.at[pl.ds(e, 1)],
                              dbuf.at[pl.ds(slot, 1)], dsem.at[slot]).start()

    for b in range(NBUF - 1):
        @pl.when(b < nu)
        def _pro():
            start_copy(b, b)

    i1r = jnp.broadcast_to(_row_of(i1, T), (E, T))
    i2r = jnp.broadcast_to(_row_of(i2, T), (E, T))
    w1r = jnp.broadcast_to(_row_of(w1, T), (E, T))
    w2r = jnp.broadcast_to(_row_of(w2, T), (E, T))
    e_iota = jax.lax.broadcasted_iota(jnp.int32, (E, T), 0)
    zero = jnp.zeros((E, T), jnp.float32)
    wv[...] = (jnp.where(e_iota == i1r, w1r, zero)
               + jnp.where(e_iota == i2r, w2r, zero))

    out_ref[...] = jnp.zeros_like(out_ref)

    def body(u, _):
        nxt = u + NBUF - 1

        @pl.when(nxt < nu)
        def _issue():
            start_copy(nxt, jax.lax.rem(nxt, NBUF))

        slot = jax.lax.rem(u, NBUF)
        e = us[0, u]
        pltpu.make_async_copy(gup_ref.at[pl.ds(e, 1)],
                              gbuf.at[pl.ds(slot, 1)], gsem.at[slot]).wait()
        pltpu.make_async_copy(dnp_ref.at[pl.ds(e, 1)],
                              dbuf.at[pl.ds(slot, 1)], dsem.at[slot]).wait()

        g = gbuf[slot]
        gu = jax.lax.dot_general(
            x_ref[...], g, (((1,), (1,)), ((), ())),
            preferred_element_type=jnp.float32)
        gate = gu[:, :I]
        up = gu[:, I:]
        act = gate * jax.nn.sigmoid(gate) * up
        dn = dbuf[slot]
        eo = jax.lax.dot_general(
            act, dn, (((1,), (1,)), ((), ())),
            preferred_element_type=jnp.float32)
        wrow = wv[pl.ds(e, 1), :]
        wcol = _col_of(wrow, T)
        out_ref[...] += wcol * eo
        return 0

    jax.lax.fori_loop(0, nu, body, 0)


@jax.jit
def _fused_moe(x, router_logits, gate_up_proj, down_proj):
    return pl.pallas_call(
        _moe_body,
        in_specs=[
            pl.BlockSpec(memory_space=pltpu.VMEM),
            pl.BlockSpec(memory_space=pltpu.VMEM),
            pl.BlockSpec(memory_space=pl.ANY),
            pl.BlockSpec(memory_space=pl.ANY),
        ],
        out_specs=pl.BlockSpec(memory_space=pltpu.VMEM),
        out_shape=jax.ShapeDtypeStruct((T, H), jnp.float32),
        scratch_shapes=[
            pltpu.VMEM((E, T), jnp.float32),
            pltpu.VMEM((1, S), jnp.int32),
            pltpu.VMEM((1, 1), jnp.int32),
            pltpu.SMEM((1, S), jnp.int32),
            pltpu.SMEM((1, 1), jnp.int32),
            pltpu.VMEM((NBUF, I2, H), jnp.float32),
            pltpu.VMEM((NBUF, H, I), jnp.float32),
            pltpu.SemaphoreType.DMA,
            pltpu.SemaphoreType.DMA,
            pltpu.SemaphoreType.DMA((NBUF,)),
            pltpu.SemaphoreType.DMA((NBUF,)),
        ],
        compiler_params=pltpu.CompilerParams(
            vmem_limit_bytes=100 * 1024 * 1024),
    )(router_logits, x, gate_up_proj, down_proj)


def kernel(x, router_logits, gate_up_proj, down_proj, top_k):
    del top_k
    return _fused_moe(x, router_logits, gate_up_proj, down_proj).astype(x.dtype)
